# Initial kernel scaffold; baseline (speedup 1.0000x reference)
import jax
import jax.numpy as jnp
from jax.experimental import pallas as pl


def kernel(query_label, color, q_feat, s_feat, object_index):
    raise NotImplementedError("write your pallas kernel here")


# speedup vs baseline: 21.0892x; 21.0892x over previous
import functools

import jax
import jax.numpy as jnp
from jax.experimental import pallas as pl
from jax.experimental.pallas import tpu as pltpu

_C = 384
_HF = 64
_WF = 64
_N = _HF * _WF
_K = 20
_SBLK = 512
_GRID = _N // _SBLK
_NITER = 20


def _topk_mean(v, kf, nb):
    lo = jnp.full((1, nb), -1.001, jnp.float32)
    hi = jnp.full((1, nb), 1.001, jnp.float32)
    for _ in range(_NITER):
        mid = 0.5 * (lo + hi)
        cnt = jnp.sum((v > mid).astype(jnp.float32), axis=0, keepdims=True)
        pred = cnt > kf
        lo = jnp.where(pred, mid, lo)
        hi = jnp.where(pred, hi, mid)
    t = hi
    gt = v > t
    cntt = jnp.sum(gt.astype(jnp.float32), axis=0, keepdims=True)
    total = jnp.sum(jnp.where(gt, v, 0.0), axis=0, keepdims=True)
    total = total + (kf - cntt) * t
    score = total / jnp.maximum(kf, 1.0)
    return jnp.where(kf > 0.0, score, 0.0)


def _body(q_ref, s_ref, m_ref, fg_ref, bg_ref, qn_ref):
    @pl.when(pl.program_id(0) == 0)
    def _():
        q = q_ref[...]
        qss = jnp.sum(q * q, axis=1, keepdims=True)
        qn_ref[...] = q * (1.0 / jnp.maximum(jnp.sqrt(qss), 1e-12))

    qn = qn_ref[...]
    s = s_ref[...]
    sss = jnp.sum(s * s, axis=1, keepdims=True)
    sn = s * (1.0 / jnp.maximum(jnp.sqrt(sss), 1e-12))
    sim = jax.lax.dot_general(
        qn, sn, (((1,), (1,)), ((), ())),
        preferred_element_type=jnp.float32)

    m = m_ref[...]
    msum = jnp.sum(m)
    kf = jnp.minimum(msum, float(_K))
    kb = jnp.minimum(float(_N) - msum, float(_K))
    simf = jnp.where(m > 0.5, sim, -3.0)
    simb = jnp.where(m > 0.5, -3.0, sim)
    fg_ref[...] = _topk_mean(simf, kf, _SBLK)
    bg_ref[...] = _topk_mean(simb, kb, _SBLK)


@functools.partial(jax.jit, static_argnames=("interpret",))
def _run(q2, s2, maskf, interpret=False):
    fg, bg = pl.pallas_call(
        _body,
        grid=(_GRID,),
        in_specs=[
            pl.BlockSpec((_N, _C), lambda i: (0, 0)),
            pl.BlockSpec((_SBLK, _C), lambda i: (i, 0)),
            pl.BlockSpec((_N, 1), lambda i: (0, 0)),
        ],
        out_specs=[
            pl.BlockSpec((1, _SBLK), lambda i: (0, i)),
            pl.BlockSpec((1, _SBLK), lambda i: (0, i)),
        ],
        out_shape=[
            jax.ShapeDtypeStruct((1, _N), jnp.float32),
            jax.ShapeDtypeStruct((1, _N), jnp.float32),
        ],
        scratch_shapes=[pltpu.VMEM((_N, _C), jnp.float32)],
        interpret=interpret,
    )(q2, s2, maskf)
    return fg, bg


def kernel(query_label, color, q_feat, s_feat, object_index):
    q2 = jnp.transpose(q_feat[0], (1, 2, 0)).reshape(_N, _C)
    s2 = jnp.transpose(s_feat[0], (1, 2, 0)).reshape(_N, _C)
    maskf = jnp.all(query_label == color, axis=-1).reshape(_N, 1)
    maskf = maskf.astype(jnp.float32)
    fg, bg = _run(q2, s2, maskf)
    return (fg.reshape(_HF, _WF), bg.reshape(_HF, _WF))
